# initial kernel scaffold (unmeasured)
import jax
import jax.numpy as jnp
from jax import lax
from jax.experimental import pallas as pl
from jax.experimental.pallas import tpu as pltpu

N_DEV = 16
H_R = 8
H_L = 7


def kernel(x, w_mat):
    m_per, k = x.shape
    _, n_per = w_mat.shape

    def body(x_ref, w_ref, out_ref, xbf, wbf, rbuf, lbuf, abuf,
             r_send, r_recv, l_send, l_recv, a_send, a_recv):
        me = lax.axis_index("i")
        left = (me + N_DEV - 1) % N_DEV
        right = (me + 1) % N_DEV

        barrier = pltpu.get_barrier_semaphore()
        for nbr in (left, right):
            pl.semaphore_signal(barrier, inc=1, device_id=(nbr,),
                                device_id_type=pl.DeviceIdType.MESH)
        pl.semaphore_wait(barrier, 2)

        xbf[...] = x_ref[...].astype(jnp.bfloat16)
        wbf[...] = w_ref[...].astype(jnp.bfloat16)

        def start_send(src, dst, ssem, rsem, dev):
            rdma = pltpu.make_async_remote_copy(
                src_ref=src, dst_ref=dst, send_sem=ssem, recv_sem=rsem,
                device_id=(dev,), device_id_type=pl.DeviceIdType.MESH)
            rdma.start()
            return rdma

        def wait_recv(dst, rsem):
            pltpu.make_async_remote_copy(
                src_ref=dst, dst_ref=dst, send_sem=rsem, recv_sem=rsem,
                device_id=(me,), device_id_type=pl.DeviceIdType.MESH,
            ).wait_recv()

        def gemm_store(chunk, origin):
            y = jnp.dot(chunk, wbf[...], preferred_element_type=jnp.float32)
            out_ref[pl.ds(origin * m_per, m_per), :] = jnp.maximum(y, 0.0)

        sends = [
            start_send(xbf, rbuf.at[0], r_send.at[0], r_recv.at[0], right),
            start_send(xbf, lbuf.at[0], l_send.at[0], l_recv.at[0], left),
        ]
        gemm_store(xbf[...], me)

        for h in range(H_R):
            wait_recv(rbuf.at[h], r_recv.at[h])
            if h + 1 < H_R:
                sends.append(start_send(rbuf.at[h], rbuf.at[h + 1],
                                        r_send.at[h + 1], r_recv.at[h + 1],
                                        right))
            if h < H_L:
                wait_recv(lbuf.at[h], l_recv.at[h])
                if h + 1 < H_L:
                    sends.append(start_send(lbuf.at[h], lbuf.at[h + 1],
                                            l_send.at[h + 1], l_recv.at[h + 1],
                                            left))
            gemm_store(rbuf[h], (me + N_DEV - 1 - h) % N_DEV)
            if h < H_L:
                gemm_store(lbuf[h], (me + 1 + h) % N_DEV)

        for s in sends:
            s.wait_send()

        local_amax = jnp.max(out_ref[...])
        abuf[0] = jnp.full(abuf.shape[1:], local_amax, jnp.float32)
        a_sends = []
        for kk in range(1, N_DEV):
            tgt = (me + kk) % N_DEV
            a_sends.append(start_send(abuf.at[0], abuf.at[N_DEV - kk],
                                      a_send.at[kk], a_recv.at[N_DEV - kk],
                                      tgt))
        for j in range(1, N_DEV):
            wait_recv(abuf.at[j], a_recv.at[j])

        global_amax = jnp.max(abuf[...])
        scale = global_amax * (1.0 / 448.0)
        q = jnp.minimum(out_ref[...] * (1.0 / scale), 448.0)
        out_ref[...] = q.astype(jnp.float8_e4m3fn).astype(jnp.float32) * scale

        for s in a_sends:
            s.wait_send()

    return pl.pallas_call(
        body,
        out_shape=jax.ShapeDtypeStruct((N_DEV * m_per, n_per), jnp.float32),
        in_specs=[pl.BlockSpec(memory_space=pltpu.VMEM),
                  pl.BlockSpec(memory_space=pltpu.VMEM)],
        out_specs=pl.BlockSpec(memory_space=pltpu.VMEM),
        scratch_shapes=[
            pltpu.VMEM((m_per, k), jnp.bfloat16),
            pltpu.VMEM((k, n_per), jnp.bfloat16),
            pltpu.VMEM((H_R, m_per, k), jnp.bfloat16),
            pltpu.VMEM((H_L, m_per, k), jnp.bfloat16),
            pltpu.VMEM((N_DEV, 8, 128), jnp.float32),
            pltpu.SemaphoreType.DMA((H_R,)),
            pltpu.SemaphoreType.DMA((H_R,)),
            pltpu.SemaphoreType.DMA((H_L,)),
            pltpu.SemaphoreType.DMA((H_L,)),
            pltpu.SemaphoreType.DMA((N_DEV,)),
            pltpu.SemaphoreType.DMA((N_DEV,)),
        ],
        compiler_params=pltpu.CompilerParams(collective_id=0),
    )(x, w_mat)


# baseline (device time: 211980 ns/iter reference)
import jax
import jax.numpy as jnp
from jax import lax
from jax.experimental import pallas as pl
from jax.experimental.pallas import tpu as pltpu

N_DEV = 16
H_R = 8
H_L = 7


def kernel(x, w_mat):
    m_per, k = x.shape
    _, n_per = w_mat.shape

    def body(x_ref, w_ref, out_ref, xbf, wbf, rbuf, lbuf, abuf,
             r_send, r_recv, l_send, l_recv, a_send, a_recv):
        me = lax.axis_index("i")
        left = (me + N_DEV - 1) % N_DEV
        right = (me + 1) % N_DEV

        barrier = pltpu.get_barrier_semaphore()
        for nbr in (left, right):
            pl.semaphore_signal(barrier, inc=1, device_id=(nbr,),
                                device_id_type=pl.DeviceIdType.MESH)
        pl.semaphore_wait(barrier, 2)

        xbf[...] = x_ref[...].astype(jnp.bfloat16)
        wbf[...] = w_ref[...].astype(jnp.bfloat16)

        def start_send(src, dst, ssem, rsem, dev):
            rdma = pltpu.make_async_remote_copy(
                src_ref=src, dst_ref=dst, send_sem=ssem, recv_sem=rsem,
                device_id=(dev,), device_id_type=pl.DeviceIdType.MESH)
            rdma.start()
            return rdma

        def wait_recv(dst, rsem):
            pltpu.make_async_remote_copy(
                src_ref=dst, dst_ref=dst, send_sem=rsem, recv_sem=rsem,
                device_id=(me,), device_id_type=pl.DeviceIdType.MESH,
            ).wait_recv()

        def gemm_store(chunk, origin):
            y = jnp.dot(chunk, wbf[...], preferred_element_type=jnp.float32)
            out_ref[pl.ds(origin * m_per, m_per), :] = jnp.maximum(y, 0.0)

        sends = [
            start_send(xbf, rbuf.at[0], r_send.at[0], r_recv.at[0], right),
            start_send(xbf, lbuf.at[0], l_send.at[0], l_recv.at[0], left),
        ]
        gemm_store(xbf[...], me)

        for h in range(H_R):
            wait_recv(rbuf.at[h], r_recv.at[h])
            if h + 1 < H_R:
                sends.append(start_send(rbuf.at[h], rbuf.at[h + 1],
                                        r_send.at[h + 1], r_recv.at[h + 1],
                                        right))
            if h < H_L:
                wait_recv(lbuf.at[h], l_recv.at[h])
                if h + 1 < H_L:
                    sends.append(start_send(lbuf.at[h], lbuf.at[h + 1],
                                            l_send.at[h + 1], l_recv.at[h + 1],
                                            left))
            gemm_store(rbuf[h], (me + N_DEV - 1 - h) % N_DEV)
            if h < H_L:
                gemm_store(lbuf[h], (me + 1 + h) % N_DEV)

        for s in sends:
            s.wait_send()

        local_amax = jnp.max(out_ref[...])
        abuf[0] = jnp.full(abuf.shape[1:], local_amax, jnp.float32)
        a_sends = []
        for kk in range(1, N_DEV):
            tgt = (me + kk) % N_DEV
            a_sends.append(start_send(abuf.at[0], abuf.at[N_DEV - kk],
                                      a_send.at[kk], a_recv.at[N_DEV - kk],
                                      tgt))
        for j in range(1, N_DEV):
            wait_recv(abuf.at[j], a_recv.at[j])

        global_amax = jnp.max(abuf[...])
        scale = global_amax * (1.0 / 448.0)
        q = jnp.minimum(out_ref[...] * (1.0 / scale), 448.0)
        out_ref[...] = q.astype(jnp.float8_e4m3fn).astype(jnp.float32) * scale

        for s in a_sends:
            s.wait_send()

    return pl.pallas_call(
        body,
        out_shape=jax.ShapeDtypeStruct((N_DEV * m_per, n_per), jnp.float32),
        in_specs=[pl.BlockSpec(memory_space=pltpu.VMEM),
                  pl.BlockSpec(memory_space=pltpu.VMEM)],
        out_specs=pl.BlockSpec(memory_space=pltpu.VMEM),
        scratch_shapes=[
            pltpu.VMEM((m_per, k), jnp.bfloat16),
            pltpu.VMEM((k, n_per), jnp.bfloat16),
            pltpu.VMEM((H_R, m_per, k), jnp.bfloat16),
            pltpu.VMEM((H_L, m_per, k), jnp.bfloat16),
            pltpu.VMEM((N_DEV, 8, 128), jnp.float32),
            pltpu.SemaphoreType.DMA((H_R,)),
            pltpu.SemaphoreType.DMA((H_R,)),
            pltpu.SemaphoreType.DMA((H_L,)),
            pltpu.SemaphoreType.DMA((H_L,)),
            pltpu.SemaphoreType.DMA((N_DEV,)),
            pltpu.SemaphoreType.DMA((N_DEV,)),
        ],
        compiler_params=pltpu.CompilerParams(
            collective_id=0, vmem_limit_bytes=100 * 1024 * 1024),
    )(x, w_mat)


# device time: 211018 ns/iter; 1.0046x vs baseline; 1.0046x over previous
import jax
import jax.numpy as jnp
from jax import lax
from jax.experimental import pallas as pl
from jax.experimental.pallas import tpu as pltpu

N_DEV = 16
H = 8


def kernel(x, w_mat):
    m_per, k = x.shape
    _, n_per = w_mat.shape

    def body(x_ref, w_ref, out_ref, xbf, wbf, rbuf, lbuf, abuf,
             r_send, r_recv, l_send, l_recv, a_send, a_recv):
        me = lax.axis_index("i")
        left = (me + N_DEV - 1) % N_DEV
        right = (me + 1) % N_DEV

        barrier = pltpu.get_barrier_semaphore()
        for nbr in (left, right):
            pl.semaphore_signal(barrier, inc=1, device_id=(nbr,),
                                device_id_type=pl.DeviceIdType.MESH)
        pl.semaphore_wait(barrier, 2)

        xbf[...] = x_ref[...].astype(jnp.bfloat16)
        wbf[...] = w_ref[...].astype(jnp.bfloat16)

        def start_send(src, dst, ssem, rsem, dev):
            rdma = pltpu.make_async_remote_copy(
                src_ref=src, dst_ref=dst, send_sem=ssem, recv_sem=rsem,
                device_id=(dev,), device_id_type=pl.DeviceIdType.MESH)
            rdma.start()
            return rdma

        def wait_recv(dst, rsem):
            pltpu.make_async_remote_copy(
                src_ref=dst, dst_ref=dst, send_sem=rsem, recv_sem=rsem,
                device_id=(me,), device_id_type=pl.DeviceIdType.MESH,
            ).wait_recv()

        half = m_per // 2

        def gemm_store(chunk, row0, rows):
            y = jnp.dot(chunk, wbf[...], preferred_element_type=jnp.float32)
            out_ref[pl.ds(row0, rows), :] = jnp.maximum(y, 0.0)

        sends = [
            start_send(xbf, rbuf.at[0], r_send.at[0], r_recv.at[0], right),
            start_send(xbf, lbuf.at[0], l_send.at[0], l_recv.at[0], left),
        ]
        gemm_store(xbf[...], me * m_per, m_per)

        for h in range(H):
            if h < H - 1:
                wait_recv(rbuf.at[h], r_recv.at[h])
                wait_recv(lbuf.at[h], l_recv.at[h])
            else:
                wait_recv(rbuf.at[h, pl.ds(0, half)], r_recv.at[h])
                wait_recv(lbuf.at[h, pl.ds(half, half)], l_recv.at[h])
            if h + 1 < H - 1:
                sends.append(start_send(rbuf.at[h], rbuf.at[h + 1],
                                        r_send.at[h + 1], r_recv.at[h + 1],
                                        right))
                sends.append(start_send(lbuf.at[h], lbuf.at[h + 1],
                                        l_send.at[h + 1], l_recv.at[h + 1],
                                        left))
            elif h + 1 == H - 1:
                sends.append(start_send(rbuf.at[h, pl.ds(0, half)],
                                        rbuf.at[h + 1, pl.ds(0, half)],
                                        r_send.at[h + 1], r_recv.at[h + 1],
                                        right))
                sends.append(start_send(lbuf.at[h, pl.ds(half, half)],
                                        lbuf.at[h + 1, pl.ds(half, half)],
                                        l_send.at[h + 1], l_recv.at[h + 1],
                                        left))
            if h < H - 1:
                gemm_store(rbuf[h], ((me + N_DEV - 1 - h) % N_DEV) * m_per,
                           m_per)
                gemm_store(lbuf[h], ((me + 1 + h) % N_DEV) * m_per, m_per)
            else:
                far = ((me + N_DEV - 8) % N_DEV) * m_per
                gemm_store(rbuf[h, :half, :], far, half)
                gemm_store(lbuf[h, half:, :], far + half, half)

        for s in sends:
            s.wait_send()

        local_amax = jnp.max(out_ref[...])
        abuf[0] = jnp.full(abuf.shape[1:], local_amax, jnp.float32)
        a_sends = []
        for kk in range(1, N_DEV):
            tgt = (me + kk) % N_DEV
            a_sends.append(start_send(abuf.at[0], abuf.at[N_DEV - kk],
                                      a_send.at[kk], a_recv.at[N_DEV - kk],
                                      tgt))
        for j in range(1, N_DEV):
            wait_recv(abuf.at[j], a_recv.at[j])

        global_amax = jnp.max(abuf[...])
        scale = global_amax * (1.0 / 448.0)
        q = jnp.minimum(out_ref[...] * (1.0 / scale), 448.0)
        out_ref[...] = q.astype(jnp.float8_e4m3fn).astype(jnp.float32) * scale

        for s in a_sends:
            s.wait_send()

    return pl.pallas_call(
        body,
        out_shape=jax.ShapeDtypeStruct((N_DEV * m_per, n_per), jnp.float32),
        in_specs=[pl.BlockSpec(memory_space=pltpu.VMEM),
                  pl.BlockSpec(memory_space=pltpu.VMEM)],
        out_specs=pl.BlockSpec(memory_space=pltpu.VMEM),
        scratch_shapes=[
            pltpu.VMEM((m_per, k), jnp.bfloat16),
            pltpu.VMEM((k, n_per), jnp.bfloat16),
            pltpu.VMEM((H, m_per, k), jnp.bfloat16),
            pltpu.VMEM((H, m_per, k), jnp.bfloat16),
            pltpu.VMEM((N_DEV, 8, 128), jnp.float32),
            pltpu.SemaphoreType.DMA((H,)),
            pltpu.SemaphoreType.DMA((H,)),
            pltpu.SemaphoreType.DMA((H,)),
            pltpu.SemaphoreType.DMA((H,)),
            pltpu.SemaphoreType.DMA((N_DEV,)),
            pltpu.SemaphoreType.DMA((N_DEV,)),
        ],
        compiler_params=pltpu.CompilerParams(
            collective_id=0, vmem_limit_bytes=100 * 1024 * 1024),
    )(x, w_mat)


# device time: 206556 ns/iter; 1.0263x vs baseline; 1.0216x over previous
import os

import jax
import jax.numpy as jnp
from jax import lax
from jax.experimental import pallas as pl
from jax.experimental.pallas import tpu as pltpu

N_DEV = 16
H = 8

_KMODE = int(os.environ.get("KMODE", "0"))


def kernel(x, w_mat):
    m_per, k = x.shape
    _, n_per = w_mat.shape

    def body(x_ref, w_ref, out_ref, xbf, wbf, rbuf, lbuf, abuf,
             r_send, r_recv, l_send, l_recv, a_send, a_recv):
        me = lax.axis_index("i")
        left = (me + N_DEV - 1) % N_DEV
        right = (me + 1) % N_DEV

        barrier = pltpu.get_barrier_semaphore()
        for nbr in (left, right):
            pl.semaphore_signal(barrier, inc=1, device_id=(nbr,),
                                device_id_type=pl.DeviceIdType.MESH)
        pl.semaphore_wait(barrier, 2)

        xbf[...] = x_ref[...].astype(jnp.bfloat16)
        wbf[...] = w_ref[...].astype(jnp.bfloat16)

        def start_send(src, dst, ssem, rsem, dev):
            rdma = pltpu.make_async_remote_copy(
                src_ref=src, dst_ref=dst, send_sem=ssem, recv_sem=rsem,
                device_id=(dev,), device_id_type=pl.DeviceIdType.MESH)
            rdma.start()
            return rdma

        def wait_recv(dst, rsem):
            pltpu.make_async_remote_copy(
                src_ref=dst, dst_ref=dst, send_sem=rsem, recv_sem=rsem,
                device_id=(me,), device_id_type=pl.DeviceIdType.MESH,
            ).wait_recv()

        half = m_per // 2

        def gemm_store(chunk, row0, rows):
            if _KMODE >= 2:
                return
            y = jnp.dot(chunk, wbf[...], preferred_element_type=jnp.float32)
            out_ref[pl.ds(row0, rows), :] = jnp.maximum(y, 0.0)

        sends = [
            start_send(xbf, rbuf.at[0], r_send.at[0], r_recv.at[0], right),
            start_send(xbf, lbuf.at[0], l_send.at[0], l_recv.at[0], left),
        ]
        gemm_store(xbf[...], me * m_per, m_per)

        for h in range(H):
            if h < H - 1:
                wait_recv(rbuf.at[h], r_recv.at[h])
                wait_recv(lbuf.at[h], l_recv.at[h])
            else:
                wait_recv(rbuf.at[h, pl.ds(0, half)], r_recv.at[h])
                wait_recv(lbuf.at[h, pl.ds(half, half)], l_recv.at[h])
            if h + 1 < H - 1:
                sends.append(start_send(rbuf.at[h], rbuf.at[h + 1],
                                        r_send.at[h + 1], r_recv.at[h + 1],
                                        right))
                sends.append(start_send(lbuf.at[h], lbuf.at[h + 1],
                                        l_send.at[h + 1], l_recv.at[h + 1],
                                        left))
            elif h + 1 == H - 1:
                sends.append(start_send(rbuf.at[h, pl.ds(0, half)],
                                        rbuf.at[h + 1, pl.ds(0, half)],
                                        r_send.at[h + 1], r_recv.at[h + 1],
                                        right))
                sends.append(start_send(lbuf.at[h, pl.ds(half, half)],
                                        lbuf.at[h + 1, pl.ds(half, half)],
                                        l_send.at[h + 1], l_recv.at[h + 1],
                                        left))
            if h < H - 1:
                gemm_store(rbuf[h], ((me + N_DEV - 1 - h) % N_DEV) * m_per,
                           m_per)
                gemm_store(lbuf[h], ((me + 1 + h) % N_DEV) * m_per, m_per)
            else:
                far = ((me + N_DEV - 8) % N_DEV) * m_per
                gemm_store(rbuf[h, :half, :], far, half)
                gemm_store(lbuf[h, half:, :], far + half, half)

        for s in sends:
            s.wait_send()

        if _KMODE >= 2:
            return

        local_amax = jnp.max(out_ref[...])
        a_sends = []
        if _KMODE == 0:
            abuf[0] = jnp.full(abuf.shape[1:], local_amax, jnp.float32)
            for kk in range(1, N_DEV):
                tgt = (me + kk) % N_DEV
                a_sends.append(start_send(abuf.at[0], abuf.at[N_DEV - kk],
                                          a_send.at[kk],
                                          a_recv.at[N_DEV - kk], tgt))
            for j in range(1, N_DEV):
                wait_recv(abuf.at[j], a_recv.at[j])
            global_amax = jnp.max(abuf[...])
        else:
            global_amax = local_amax

        scale = global_amax * (1.0 / 448.0)
        q = jnp.minimum(out_ref[...] * (1.0 / scale), 448.0)
        out_ref[...] = q.astype(jnp.float8_e4m3fn).astype(jnp.float32) * scale

        for s in a_sends:
            s.wait_send()

    return pl.pallas_call(
        body,
        out_shape=jax.ShapeDtypeStruct((N_DEV * m_per, n_per), jnp.float32),
        in_specs=[pl.BlockSpec(memory_space=pltpu.VMEM),
                  pl.BlockSpec(memory_space=pltpu.VMEM)],
        out_specs=pl.BlockSpec(memory_space=pltpu.VMEM),
        scratch_shapes=[
            pltpu.VMEM((m_per, k), jnp.bfloat16),
            pltpu.VMEM((k, n_per), jnp.bfloat16),
            pltpu.VMEM((H, m_per, k), jnp.bfloat16),
            pltpu.VMEM((H, m_per, k), jnp.bfloat16),
            pltpu.VMEM((N_DEV, 8, 128), jnp.float32),
            pltpu.SemaphoreType.DMA((H,)),
            pltpu.SemaphoreType.DMA((H,)),
            pltpu.SemaphoreType.DMA((H,)),
            pltpu.SemaphoreType.DMA((H,)),
            pltpu.SemaphoreType.DMA((N_DEV,)),
            pltpu.SemaphoreType.DMA((N_DEV,)),
        ],
        compiler_params=pltpu.CompilerParams(
            collective_id=0, vmem_limit_bytes=100 * 1024 * 1024),
    )(x, w_mat)


# device time: 190456 ns/iter; 1.1130x vs baseline; 1.0845x over previous
import os

import jax
import jax.numpy as jnp
from jax import lax
from jax.experimental import pallas as pl
from jax.experimental.pallas import tpu as pltpu

N_DEV = 16
H = 8
SUBS = 2

_KMODE = int(os.environ.get("KMODE", "0"))


def kernel(x, w_mat):
    m_per, k = x.shape
    _, n_per = w_mat.shape

    def body(x_ref, w_ref, out_ref, xbf, wbf, rbuf, lbuf, abuf,
             r_send, r_recv, l_send, l_recv, a_send, a_recv):
        me = lax.axis_index("i")
        left = (me + N_DEV - 1) % N_DEV
        right = (me + 1) % N_DEV

        barrier = pltpu.get_barrier_semaphore()
        for nbr in (left, right):
            pl.semaphore_signal(barrier, inc=1, device_id=(nbr,),
                                device_id_type=pl.DeviceIdType.MESH)
        pl.semaphore_wait(barrier, 2)

        xbf[...] = x_ref[...].astype(jnp.bfloat16)
        wbf[...] = w_ref[...].astype(jnp.bfloat16)

        def start_send(src, dst, ssem, rsem, dev):
            rdma = pltpu.make_async_remote_copy(
                src_ref=src, dst_ref=dst, send_sem=ssem, recv_sem=rsem,
                device_id=(dev,), device_id_type=pl.DeviceIdType.MESH)
            rdma.start()
            return rdma

        def wait_recv(dst, rsem):
            pltpu.make_async_remote_copy(
                src_ref=dst, dst_ref=dst, send_sem=rsem, recv_sem=rsem,
                device_id=(me,), device_id_type=pl.DeviceIdType.MESH,
            ).wait_recv()

        half = m_per // 2

        def gemm_store(chunk, row0, rows):
            if _KMODE >= 2:
                return
            y = jnp.dot(chunk, wbf[...], preferred_element_type=jnp.float32)
            out_ref[pl.ds(row0, rows), :] = jnp.maximum(y, 0.0)

        def sub_rows(h, s, is_left):
            if h < H - 1:
                return ((SUBS - 1 - s) if is_left else s) * half, half
            q = half // 2
            return (half if is_left else 0) + s * q, q

        sends = []
        for s in range(SUBS):
            r0, nr = sub_rows(0, s, False)
            sends.append(start_send(xbf.at[pl.ds(r0, nr)],
                                    rbuf.at[0, pl.ds(r0, nr)],
                                    r_send.at[0, s], r_recv.at[0, s], right))
            r0, nr = sub_rows(0, s, True)
            sends.append(start_send(xbf.at[pl.ds(r0, nr)],
                                    lbuf.at[0, pl.ds(r0, nr)],
                                    l_send.at[0, s], l_recv.at[0, s], left))
        gemm_store(xbf[...], me * m_per, m_per)

        for h in range(H):
            for s in range(SUBS):
                rr0, rnr = sub_rows(h, s, False)
                lr0, lnr = sub_rows(h, s, True)
                wait_recv(rbuf.at[h, pl.ds(rr0, rnr)], r_recv.at[h, s])
                if h + 1 < H:
                    fr0, fnr = sub_rows(h + 1, s, False)
                    sends.append(start_send(
                        rbuf.at[h, pl.ds(fr0, fnr)],
                        rbuf.at[h + 1, pl.ds(fr0, fnr)],
                        r_send.at[h + 1, s], r_recv.at[h + 1, s], right))
                wait_recv(lbuf.at[h, pl.ds(lr0, lnr)], l_recv.at[h, s])
                if h + 1 < H:
                    fr0, fnr = sub_rows(h + 1, s, True)
                    sends.append(start_send(
                        lbuf.at[h, pl.ds(fr0, fnr)],
                        lbuf.at[h + 1, pl.ds(fr0, fnr)],
                        l_send.at[h + 1, s], l_recv.at[h + 1, s], left))
            if h < H - 1:
                gemm_store(rbuf[h], ((me + N_DEV - 1 - h) % N_DEV) * m_per,
                           m_per)
                gemm_store(lbuf[h], ((me + 1 + h) % N_DEV) * m_per, m_per)
            else:
                far = ((me + N_DEV - 8) % N_DEV) * m_per
                gemm_store(rbuf[h, :half, :], far, half)
                gemm_store(lbuf[h, half:, :], far + half, half)

        for s in sends:
            s.wait_send()

        if _KMODE >= 2:
            return

        local_amax = jnp.max(out_ref[...])
        a_sends = []
        if _KMODE == 0:
            abuf[0] = jnp.full(abuf.shape[1:], local_amax, jnp.float32)
            for kk in range(1, N_DEV):
                tgt = (me + kk) % N_DEV
                a_sends.append(start_send(abuf.at[0], abuf.at[N_DEV - kk],
                                          a_send.at[kk],
                                          a_recv.at[N_DEV - kk], tgt))
            for j in range(1, N_DEV):
                wait_recv(abuf.at[j], a_recv.at[j])
            global_amax = jnp.max(abuf[...])
        else:
            global_amax = local_amax

        scale = global_amax * (1.0 / 448.0)
        q = jnp.minimum(out_ref[...] * (1.0 / scale), 448.0)
        out_ref[...] = q.astype(jnp.float8_e4m3fn).astype(jnp.float32) * scale

        for s in a_sends:
            s.wait_send()

    return pl.pallas_call(
        body,
        out_shape=jax.ShapeDtypeStruct((N_DEV * m_per, n_per), jnp.float32),
        in_specs=[pl.BlockSpec(memory_space=pltpu.VMEM),
                  pl.BlockSpec(memory_space=pltpu.VMEM)],
        out_specs=pl.BlockSpec(memory_space=pltpu.VMEM),
        scratch_shapes=[
            pltpu.VMEM((m_per, k), jnp.bfloat16),
            pltpu.VMEM((k, n_per), jnp.bfloat16),
            pltpu.VMEM((H, m_per, k), jnp.bfloat16),
            pltpu.VMEM((H, m_per, k), jnp.bfloat16),
            pltpu.VMEM((N_DEV, 8, 128), jnp.float32),
            pltpu.SemaphoreType.DMA((H, SUBS)),
            pltpu.SemaphoreType.DMA((H, SUBS)),
            pltpu.SemaphoreType.DMA((H, SUBS)),
            pltpu.SemaphoreType.DMA((H, SUBS)),
            pltpu.SemaphoreType.DMA((N_DEV,)),
            pltpu.SemaphoreType.DMA((N_DEV,)),
        ],
        compiler_params=pltpu.CompilerParams(
            collective_id=0, vmem_limit_bytes=100 * 1024 * 1024),
    )(x, w_mat)
